# Initial kernel scaffold; baseline (speedup 1.0000x reference)
import jax
import jax.numpy as jnp
from jax.experimental import pallas as pl


def kernel(t, gamma):
    raise NotImplementedError("write your pallas kernel here")


# speedup vs baseline: 4.5168x; 4.5168x over previous
import functools

import jax
import jax.numpy as jnp
from jax import lax
from jax.experimental import pallas as pl
from jax.experimental.pallas import tpu as pltpu
from jax.experimental.pallas import tpu_sc as plsc

_TIMESTEPS = 1000
_N = 16384
_LANES = 16
_NUM_CORES = 2
_NUM_SUBCORES = 16
_NUM_WORKERS = _NUM_CORES * _NUM_SUBCORES
_CHUNK = _N // _NUM_WORKERS
_TABLE_PAD = 1024
_MAGIC = 2.0 ** 23


@functools.partial(
    pl.kernel,
    out_type=jax.ShapeDtypeStruct((_N,), jnp.float32),
    mesh=plsc.VectorSubcoreMesh(core_axis_name="c", subcore_axis_name="s"),
    compiler_params=pltpu.CompilerParams(needs_layout_passes=False),
    scratch_types=[
        pltpu.VMEM((_TABLE_PAD,), jnp.float32),
        pltpu.VMEM((_CHUNK,), jnp.float32),
        pltpu.VMEM((_CHUNK,), jnp.float32),
    ],
)
def _gamma_lookup(t_hbm, gamma_hbm, out_hbm, gamma_v, t_v, out_v):
    wid = lax.axis_index("s") * _NUM_CORES + lax.axis_index("c")
    base = wid * _CHUNK
    pltpu.sync_copy(gamma_hbm, gamma_v)
    pltpu.sync_copy(t_hbm.at[pl.ds(base, _CHUNK)], t_v)
    for i in range(_CHUNK // _LANES):
        tv = t_v[pl.ds(i * _LANES, _LANES)]
        y = tv * jnp.float32(_TIMESTEPS)
        r = (y + jnp.float32(_MAGIC)) - jnp.float32(_MAGIC)
        idx = r.astype(jnp.int32)
        out_v[pl.ds(i * _LANES, _LANES)] = plsc.load_gather(gamma_v, [idx])
    pltpu.sync_copy(out_v, out_hbm.at[pl.ds(base, _CHUNK)])


def kernel(t, gamma):
    gamma_p = jnp.concatenate(
        [gamma, jnp.zeros((_TABLE_PAD - gamma.shape[0],), gamma.dtype)]
    )
    return _gamma_lookup(t, gamma_p)
